# baseline (speedup 1.0000x reference)
import functools

import jax
import jax.numpy as jnp
from jax import lax
from jax.experimental import pallas as pl
from jax.experimental.pallas import tpu as pltpu
from jax.experimental.pallas import tpu_sc as plsc

_ROWS = 128
_LANES = 16


def _sc_gather_fn(B, DEG, F, NC, NS):
    NW = NC * NS
    BPW = B // NW
    NPC = _ROWS // DEG
    NCHUNK = (BPW * DEG) // _ROWS
    NF = F // _LANES
    SELF_CHUNKS = BPW // _ROWS

    mesh = plsc.VectorSubcoreMesh(core_axis_name="c", subcore_axis_name="s")

    @functools.partial(
        pl.kernel,
        out_type=(jax.ShapeDtypeStruct((B, F), jnp.float32),
                  jax.ShapeDtypeStruct((B, F), jnp.float32)),
        mesh=mesh,
        scratch_types=[
            pltpu.VMEM((BPW,), jnp.int32),
            pltpu.VMEM((BPW, DEG), jnp.int32),
            pltpu.VMEM((NCHUNK, _ROWS), jnp.int32),
            pltpu.VMEM((_ROWS, F), jnp.float32),
            pltpu.VMEM((_ROWS, F), jnp.float32),
            pltpu.VMEM((_ROWS, F), jnp.float32),
            pltpu.VMEM((BPW, F), jnp.float32),
            pltpu.SemaphoreType.DMA,
            pltpu.SemaphoreType.DMA,
            pltpu.SemaphoreType.DMA,
        ],
    )
    def k(feat_hbm, ni_hbm, bn_hbm, self_hbm, nsum_hbm, bn_v, slab_v, ni_v,
          buf0, buf1, buf2, acc_v, sem0, sem1, sem2):
        wid = lax.axis_index("s") * NC + lax.axis_index("c")
        base = wid * BPW
        bufs = (buf0, buf1, buf2)
        sems = (sem0, sem1, sem2)

        pltpu.sync_copy(bn_hbm.at[pl.ds(base, BPW)], bn_v)
        pltpu.sync_copy(ni_hbm.at[pl.ds(base, BPW)], slab_v)

        for c in range(SELF_CHUNKS):
            pltpu.async_copy(feat_hbm.at[bn_v.at[pl.ds(c * _ROWS, _ROWS)]],
                             bufs[c], sems[c])

        @pl.loop(0, NCHUNK)
        def _(kk):
            for j in range(NPC):
                for c2 in range(DEG // _LANES):
                    ni_v[kk, pl.ds(j * DEG + c2 * _LANES, _LANES)] = (
                        slab_v[kk * NPC + j, pl.ds(c2 * _LANES, _LANES)])

        for c in range(SELF_CHUNKS):
            pltpu.make_async_copy(feat_hbm.at[bn_v.at[pl.ds(c * _ROWS, _ROWS)]],
                                  bufs[c], sems[c]).wait()
            pltpu.sync_copy(bufs[c], self_hbm.at[pl.ds(base + c * _ROWS,
                                                       _ROWS)])
            pltpu.async_copy(feat_hbm.at[ni_v.at[c]], bufs[c], sems[c])
        pltpu.async_copy(feat_hbm.at[ni_v.at[2]], bufs[2], sems[2])

        @pl.loop(0, NCHUNK + 2, step=3)
        def _(g):
            for b in range(3):
                chunk = g + b

                @pl.when(chunk < NCHUNK)
                def _(chunk=chunk, b=b):
                    buf = bufs[b]
                    sem = sems[b]
                    pltpu.make_async_copy(feat_hbm.at[ni_v.at[chunk]], buf,
                                          sem).wait()
                    for j in range(NPC):
                        rb = j * DEG

                        @pl.loop(
                            0, DEG,
                            init_carry=tuple(
                                jnp.zeros((_LANES,), jnp.float32)
                                for _ in range(NF)),
                            unroll=8)
                        def accs(r, carry, rb=rb, buf=buf):
                            return tuple(
                                carry[f] +
                                buf[rb + r, pl.ds(f * _LANES, _LANES)]
                                for f in range(NF))

                        node = chunk * NPC + j
                        for f in range(NF):
                            acc_v[node, pl.ds(f * _LANES, _LANES)] = accs[f]

                    @pl.when(chunk + 3 < NCHUNK)
                    def _(buf=buf, sem=sem, chunk=chunk):
                        pltpu.async_copy(feat_hbm.at[ni_v.at[chunk + 3]], buf,
                                         sem)

        pltpu.sync_copy(acc_v, nsum_hbm.at[pl.ds(base, BPW)])

    return k


def _tc_head_fn(B, DEG, F, H, C, BLK):
    inv_deg = 1.0 / DEG

    def body(s_ref, n_ref, w1_ref, w2_ref, wc_ref, o_ref):
        x = jnp.dot(s_ref[...], w1_ref[...],
                    preferred_element_type=jnp.float32)
        x = x + jnp.dot(n_ref[...] * inv_deg, w2_ref[...],
                        preferred_element_type=jnp.float32)
        h = jnp.maximum(x, 0.0)
        o_ref[...] = jnp.dot(h, wc_ref[...], preferred_element_type=jnp.float32)

    return pl.pallas_call(
        body,
        grid=(B // BLK,),
        in_specs=[
            pl.BlockSpec((BLK, F), lambda i: (i, 0)),
            pl.BlockSpec((BLK, F), lambda i: (i, 0)),
            pl.BlockSpec((F, H), lambda i: (0, 0)),
            pl.BlockSpec((F, H), lambda i: (0, 0)),
            pl.BlockSpec((H, C), lambda i: (0, 0)),
        ],
        out_specs=pl.BlockSpec((BLK, C), lambda i: (i, 0)),
        out_shape=jax.ShapeDtypeStruct((B, C), jnp.float32),
        compiler_params=pltpu.CompilerParams(
            dimension_semantics=("arbitrary",)),
    )


def kernel(features, neigh_idx, batch_nodes, W_enc, W_cls):
    B, DEG = neigh_idx.shape
    N, F = features.shape
    H = W_enc.shape[1]
    C = W_cls.shape[1]

    info = plsc.get_sparse_core_info()
    NC, NS = info.num_cores, info.num_subcores

    ni = neigh_idx.astype(jnp.int32)
    bn = batch_nodes.astype(jnp.int32)

    self32, nsum32 = _sc_gather_fn(B, DEG, F, NC, NS)(features, ni, bn)
    scores = _tc_head_fn(B, DEG, F, H, C, BLK=8192)(
        self32, nsum32, W_enc[:F], W_enc[F:], W_cls)
    return scores


# speedup vs baseline: 1.0103x; 1.0103x over previous
import functools

import jax
import jax.numpy as jnp
from jax import lax
from jax.experimental import pallas as pl
from jax.experimental.pallas import tpu as pltpu
from jax.experimental.pallas import tpu_sc as plsc

_ROWS = 128
_LANES = 16


def _sc_gather_fn(B, DEG, F, NC, NS):
    NW = NC * NS
    BPW = B // NW
    NPC = _ROWS // DEG
    NCHUNK = (BPW * DEG) // _ROWS
    NF = F // _LANES
    SELF_CHUNKS = BPW // _ROWS

    mesh = plsc.VectorSubcoreMesh(core_axis_name="c", subcore_axis_name="s")

    @functools.partial(
        pl.kernel,
        out_type=(jax.ShapeDtypeStruct((B, F), jnp.float32),
                  jax.ShapeDtypeStruct((B, F), jnp.float32)),
        mesh=mesh,
        scratch_types=[
            pltpu.VMEM((BPW,), jnp.int32),
            pltpu.VMEM((BPW, DEG), jnp.int32),
            pltpu.VMEM((NCHUNK, _ROWS), jnp.int32),
            pltpu.VMEM((_ROWS, F), jnp.float32),
            pltpu.VMEM((_ROWS, F), jnp.float32),
            pltpu.VMEM((_ROWS, F), jnp.float32),
            pltpu.VMEM((BPW, F), jnp.float32),
            pltpu.SemaphoreType.DMA,
            pltpu.SemaphoreType.DMA,
            pltpu.SemaphoreType.DMA,
            pltpu.SemaphoreType.DMA,
        ],
    )
    def k(feat_hbm, ni_hbm, bn_hbm, self_hbm, nsum_hbm, bn_v, slab_v, ni_v,
          buf0, buf1, buf2, acc_v, sem0, sem1, sem2, osem):
        wid = lax.axis_index("s") * NC + lax.axis_index("c")
        base = wid * BPW
        bufs = (buf0, buf1, buf2)
        sems = (sem0, sem1, sem2)

        pltpu.sync_copy(bn_hbm.at[pl.ds(base, BPW)], bn_v)
        for c in range(SELF_CHUNKS):
            pltpu.async_copy(feat_hbm.at[bn_v.at[pl.ds(c * _ROWS, _ROWS)]],
                             bufs[c], sems[c])
        pltpu.sync_copy(ni_hbm.at[pl.ds(base, BPW)], slab_v)

        @pl.loop(0, NCHUNK)
        def _(kk):
            for j in range(NPC):
                for c2 in range(DEG // _LANES):
                    ni_v[kk, pl.ds(j * DEG + c2 * _LANES, _LANES)] = (
                        slab_v[kk * NPC + j, pl.ds(c2 * _LANES, _LANES)])

        for c in range(SELF_CHUNKS):
            pltpu.make_async_copy(feat_hbm.at[bn_v.at[pl.ds(c * _ROWS, _ROWS)]],
                                  bufs[c], sems[c]).wait()
            pltpu.sync_copy(bufs[c], self_hbm.at[pl.ds(base + c * _ROWS,
                                                       _ROWS)])
            pltpu.async_copy(feat_hbm.at[ni_v.at[c]], bufs[c], sems[c])
        pltpu.async_copy(feat_hbm.at[ni_v.at[2]], bufs[2], sems[2])

        @pl.loop(0, NCHUNK + 2, step=3)
        def _(g):
            for b in range(3):
                chunk = g + b

                @pl.when(chunk < NCHUNK)
                def _(chunk=chunk, b=b):
                    buf = bufs[b]
                    sem = sems[b]
                    pltpu.make_async_copy(feat_hbm.at[ni_v.at[chunk]], buf,
                                          sem).wait()
                    for j in range(NPC):
                        rb = j * DEG

                        @pl.loop(
                            0, DEG,
                            init_carry=tuple(
                                jnp.zeros((_LANES,), jnp.float32)
                                for _ in range(NF)),
                            unroll=8)
                        def accs(r, carry, rb=rb, buf=buf):
                            return tuple(
                                carry[f] +
                                buf[rb + r, pl.ds(f * _LANES, _LANES)]
                                for f in range(NF))

                        node = chunk * NPC + j
                        for f in range(NF):
                            acc_v[node, pl.ds(f * _LANES, _LANES)] = accs[f]

                    pltpu.async_copy(
                        acc_v.at[pl.ds(chunk * NPC, NPC)],
                        nsum_hbm.at[pl.ds(base + chunk * NPC, NPC)], osem)

                    @pl.when(chunk + 3 < NCHUNK)
                    def _(buf=buf, sem=sem, chunk=chunk):
                        pltpu.async_copy(feat_hbm.at[ni_v.at[chunk + 3]], buf,
                                         sem)

        pltpu.make_async_copy(nsum_hbm.at[pl.ds(base, BPW)], acc_v,
                              osem).wait()

    return k


def _tc_head_fn(B, DEG, F, H, C, BLK):
    inv_deg = 1.0 / DEG

    def body(s_ref, n_ref, w1_ref, w2_ref, wc_ref, o_ref):
        x = jnp.dot(s_ref[...], w1_ref[...],
                    preferred_element_type=jnp.float32)
        x = x + jnp.dot(n_ref[...] * inv_deg, w2_ref[...],
                        preferred_element_type=jnp.float32)
        h = jnp.maximum(x, 0.0)
        o_ref[...] = jnp.dot(h, wc_ref[...], preferred_element_type=jnp.float32)

    return pl.pallas_call(
        body,
        grid=(B // BLK,),
        in_specs=[
            pl.BlockSpec((BLK, F), lambda i: (i, 0)),
            pl.BlockSpec((BLK, F), lambda i: (i, 0)),
            pl.BlockSpec((F, H), lambda i: (0, 0)),
            pl.BlockSpec((F, H), lambda i: (0, 0)),
            pl.BlockSpec((H, C), lambda i: (0, 0)),
        ],
        out_specs=pl.BlockSpec((BLK, C), lambda i: (i, 0)),
        out_shape=jax.ShapeDtypeStruct((B, C), jnp.float32),
        compiler_params=pltpu.CompilerParams(
            dimension_semantics=("arbitrary",)),
    )


def kernel(features, neigh_idx, batch_nodes, W_enc, W_cls):
    B, DEG = neigh_idx.shape
    N, F = features.shape
    H = W_enc.shape[1]
    C = W_cls.shape[1]

    info = plsc.get_sparse_core_info()
    NC, NS = info.num_cores, info.num_subcores

    ni = neigh_idx.astype(jnp.int32)
    bn = batch_nodes.astype(jnp.int32)

    self32, nsum32 = _sc_gather_fn(B, DEG, F, NC, NS)(features, ni, bn)
    scores = _tc_head_fn(B, DEG, F, H, C, BLK=4096)(
        self32, nsum32, W_enc[:F], W_enc[F:], W_cls)
    return scores
